# baseline (device time: 763702 ns/iter reference)
import jax
import jax.numpy as jnp
from jax import lax
from jax.experimental import pallas as pl
from jax.experimental.pallas import tpu as pltpu

N_DEV = 16


def _snap_e4m3(v):
    q = jnp.clip(v, -448.0, 448.0).astype(jnp.float8_e4m3fn)
    return q.astype(jnp.float32)


def kernel(x, w_mat):
    m_per, k = x.shape
    _, n_per = w_mat.shape

    def body(x_ref, w_ref, out_ref, comm_ref, maxima_ref,
             send_sems, recv_sems, credit_sem, amax_send_sems, amax_recv_sems):
        my = lax.axis_index("i")
        left = lax.rem(my - 1 + N_DEV, N_DEV)
        right = lax.rem(my + 1, N_DEV)

        barrier_sem = pltpu.get_barrier_semaphore()
        for nbr in (left, right):
            pl.semaphore_signal(barrier_sem, inc=1, device_id=(nbr,),
                                device_id_type=pl.DeviceIdType.MESH)
        pl.semaphore_wait(barrier_sem, 2)

        comm_ref[0, :, :] = x_ref[:, :]
        y = jnp.dot(x_ref[:, :], w_ref[:, :],
                    preferred_element_type=jnp.float32)
        out_ref[pl.ds(my * m_per, m_per), :] = y
        amax = jnp.max(jnp.abs(y))

        for h in range(N_DEV - 1):
            send_slot = h % 2
            recv_slot = (h + 1) % 2
            if h >= 1:
                pl.semaphore_wait(credit_sem, 1)
            rdma = pltpu.make_async_remote_copy(
                src_ref=comm_ref.at[send_slot],
                dst_ref=comm_ref.at[recv_slot],
                send_sem=send_sems.at[send_slot],
                recv_sem=recv_sems.at[recv_slot],
                device_id=(right,),
                device_id_type=pl.DeviceIdType.MESH,
            )
            rdma.start()
            rdma.wait()
            if h < N_DEV - 2:
                pl.semaphore_signal(credit_sem, inc=1, device_id=(left,),
                                    device_id_type=pl.DeviceIdType.MESH)
            origin = lax.rem(my - h - 1 + N_DEV, N_DEV)
            y = jnp.dot(comm_ref[recv_slot, :, :], w_ref[:, :],
                        preferred_element_type=jnp.float32)
            out_ref[pl.ds(origin * m_per, m_per), :] = y
            amax = jnp.maximum(amax, jnp.max(jnp.abs(y)))

        maxima_ref[pl.ds(my, 1), :] = jnp.full((1, 128), amax, jnp.float32)
        amax_rdmas = []
        for j in range(1, N_DEV):
            tgt = lax.rem(my + j, N_DEV)
            r = pltpu.make_async_remote_copy(
                src_ref=maxima_ref.at[pl.ds(my, 1)],
                dst_ref=maxima_ref.at[pl.ds(my, 1)],
                send_sem=amax_send_sems.at[j - 1],
                recv_sem=amax_recv_sems.at[j - 1],
                device_id=(tgt,),
                device_id_type=pl.DeviceIdType.MESH,
            )
            r.start()
            amax_rdmas.append(r)
        for r in amax_rdmas:
            r.wait_send()
        for r in amax_rdmas:
            r.wait_recv()

        gmax = jnp.max(maxima_ref[:, :])
        scale = gmax / 448.0
        out_ref[:, :] = _snap_e4m3(out_ref[:, :] / scale) * scale

    return pl.pallas_call(
        body,
        out_shape=jax.ShapeDtypeStruct((N_DEV * m_per, n_per), jnp.float32),
        in_specs=[
            pl.BlockSpec(memory_space=pltpu.VMEM),
            pl.BlockSpec(memory_space=pltpu.VMEM),
        ],
        out_specs=pl.BlockSpec(memory_space=pltpu.VMEM),
        scratch_shapes=[
            pltpu.VMEM((2, m_per, k), jnp.float32),
            pltpu.VMEM((N_DEV, 128), jnp.float32),
            pltpu.SemaphoreType.DMA((2,)),
            pltpu.SemaphoreType.DMA((2,)),
            pltpu.SemaphoreType.REGULAR,
            pltpu.SemaphoreType.DMA((N_DEV - 1,)),
            pltpu.SemaphoreType.DMA((N_DEV - 1,)),
        ],
        compiler_params=pltpu.CompilerParams(collective_id=0),
    )(x, w_mat)


# device time: 396594 ns/iter; 1.9257x vs baseline; 1.9257x over previous
import jax
import jax.numpy as jnp
from jax import lax
from jax.experimental import pallas as pl
from jax.experimental.pallas import tpu as pltpu

N_DEV = 16

A_CYCLE = (0, 4, 8, 12, 13, 9, 5, 1, 2, 6, 10, 14, 15, 11, 7, 3)
POS_A = tuple(A_CYCLE.index(p) for p in range(N_DEV))

HOPS_A = 8
HOPS_B = 7


def _snap_e4m3(v):
    q = jnp.clip(v, -448.0, 448.0).astype(jnp.float8_e4m3fn)
    return q.astype(jnp.float32)


def kernel(x, w_mat):
    m_per, k = x.shape
    _, n_per = w_mat.shape

    def body(cyc_ref, pos_ref, x_ref, w_ref, out_ref, comm_a, comm_b,
             maxima_ref, send_a, recv_a, send_b, recv_b, credit_a, credit_b,
             amax_send_sems, amax_recv_sems):
        my = lax.axis_index("i")
        pos = pos_ref[my]
        succ_a = cyc_ref[lax.rem(pos + 1, N_DEV)]
        pred_a = cyc_ref[lax.rem(pos - 1 + N_DEV, N_DEV)]
        succ_b = pred_a
        pred_b = succ_a

        barrier_sem = pltpu.get_barrier_semaphore()
        for nbr in (succ_a, pred_a):
            pl.semaphore_signal(barrier_sem, inc=1, device_id=(nbr,),
                                device_id_type=pl.DeviceIdType.MESH)
        pl.semaphore_wait(barrier_sem, 2)

        comm_a[0, :, :] = x_ref[:, :]
        comm_b[0, :, :] = x_ref[:, :]

        def gemm(src):
            return jnp.dot(src, w_ref[:, :],
                           preferred_element_type=jnp.float32,
                           precision=lax.Precision.HIGHEST)

        def store(origin, y):
            out_ref[pl.ds(origin * m_per, m_per), :] = y

        def make_hop(comm, send_sems, recv_sems, h, tgt):
            return pltpu.make_async_remote_copy(
                src_ref=comm.at[h % 2],
                dst_ref=comm.at[(h + 1) % 2],
                send_sem=send_sems.at[h % 2],
                recv_sem=recv_sems.at[(h + 1) % 2],
                device_id=(tgt,),
                device_id_type=pl.DeviceIdType.MESH,
            )

        amax = jnp.float32(0.0)
        for h in range(HOPS_A):
            if h >= 1:
                pl.semaphore_wait(credit_a, 1)
            rdma_a = make_hop(comm_a, send_a, recv_a, h, succ_a)
            rdma_a.start()
            rdma_b = None
            if h < HOPS_B:
                if h >= 1:
                    pl.semaphore_wait(credit_b, 1)
                rdma_b = make_hop(comm_b, send_b, recv_b, h, succ_b)
                rdma_b.start()

            if h == 0:
                y = gemm(x_ref[:, :])
                store(my, y)
                amax = jnp.max(jnp.abs(y))
            else:
                y = gemm(comm_a[h % 2, :, :])
                store(cyc_ref[lax.rem(pos - h + N_DEV, N_DEV)], y)
                amax = jnp.maximum(amax, jnp.max(jnp.abs(y)))
                if h <= HOPS_B:
                    y = gemm(comm_b[h % 2, :, :])
                    store(cyc_ref[lax.rem(pos + h, N_DEV)], y)
                    amax = jnp.maximum(amax, jnp.max(jnp.abs(y)))

            rdma_a.wait_send()
            if h < HOPS_A - 1:
                pl.semaphore_signal(credit_a, inc=1, device_id=(pred_a,),
                                    device_id_type=pl.DeviceIdType.MESH)
            if rdma_b is not None:
                rdma_b.wait_send()
                if h < HOPS_B - 1:
                    pl.semaphore_signal(credit_b, inc=1, device_id=(pred_b,),
                                        device_id_type=pl.DeviceIdType.MESH)
            rdma_a.wait_recv()
            if rdma_b is not None:
                rdma_b.wait_recv()

        y = gemm(comm_a[HOPS_A % 2, :, :])
        store(cyc_ref[lax.rem(pos - HOPS_A + N_DEV, N_DEV)], y)
        amax = jnp.maximum(amax, jnp.max(jnp.abs(y)))
        y = gemm(comm_b[HOPS_B % 2, :, :])
        store(cyc_ref[lax.rem(pos + HOPS_B, N_DEV)], y)
        amax = jnp.maximum(amax, jnp.max(jnp.abs(y)))

        maxima_ref[pl.ds(my, 1), :] = jnp.full((1, 128), amax, jnp.float32)
        amax_rdmas = []
        for j in range(1, N_DEV):
            tgt = lax.rem(my + j, N_DEV)
            r = pltpu.make_async_remote_copy(
                src_ref=maxima_ref.at[pl.ds(my, 1)],
                dst_ref=maxima_ref.at[pl.ds(my, 1)],
                send_sem=amax_send_sems.at[j - 1],
                recv_sem=amax_recv_sems.at[j - 1],
                device_id=(tgt,),
                device_id_type=pl.DeviceIdType.MESH,
            )
            r.start()
            amax_rdmas.append(r)
        for r in amax_rdmas:
            r.wait_send()
        for r in amax_rdmas:
            r.wait_recv()

        gmax = jnp.max(maxima_ref[:, :])
        scale = gmax / 448.0
        out_ref[:, :] = _snap_e4m3(out_ref[:, :] / scale) * scale

    return pl.pallas_call(
        body,
        out_shape=jax.ShapeDtypeStruct((N_DEV * m_per, n_per), jnp.float32),
        in_specs=[
            pl.BlockSpec(memory_space=pltpu.SMEM),
            pl.BlockSpec(memory_space=pltpu.SMEM),
            pl.BlockSpec(memory_space=pltpu.VMEM),
            pl.BlockSpec(memory_space=pltpu.VMEM),
        ],
        out_specs=pl.BlockSpec(memory_space=pltpu.VMEM),
        scratch_shapes=[
            pltpu.VMEM((2, m_per, k), jnp.float32),
            pltpu.VMEM((2, m_per, k), jnp.float32),
            pltpu.VMEM((N_DEV, 128), jnp.float32),
            pltpu.SemaphoreType.DMA((2,)),
            pltpu.SemaphoreType.DMA((2,)),
            pltpu.SemaphoreType.DMA((2,)),
            pltpu.SemaphoreType.DMA((2,)),
            pltpu.SemaphoreType.REGULAR,
            pltpu.SemaphoreType.REGULAR,
            pltpu.SemaphoreType.DMA((N_DEV - 1,)),
            pltpu.SemaphoreType.DMA((N_DEV - 1,)),
        ],
        compiler_params=pltpu.CompilerParams(collective_id=0),
    )(jnp.asarray(A_CYCLE, jnp.int32), jnp.asarray(POS_A, jnp.int32),
      x, w_mat)


# device time: 373870 ns/iter; 2.0427x vs baseline; 1.0608x over previous
import jax
import jax.numpy as jnp
from jax import lax
from jax.experimental import pallas as pl
from jax.experimental.pallas import tpu as pltpu

N_DEV = 16

A_CYCLE = (0, 4, 8, 12, 13, 9, 5, 1, 2, 6, 10, 14, 15, 11, 7, 3)
POS_A = tuple(A_CYCLE.index(p) for p in range(N_DEV))

HOPS_A = 8
HOPS_B = 8


def _snap_e4m3(v):
    q = jnp.clip(v, -448.0, 448.0).astype(jnp.float8_e4m3fn)
    return q.astype(jnp.float32)


def kernel(x, w_mat):
    m_per, k = x.shape
    _, n_per = w_mat.shape

    def body(cyc_ref, pos_ref, x_ref, w_ref, out_ref, comm_a, comm_b,
             maxima_ref, send_a, recv_a, send_b, recv_b, credit_a, credit_b,
             amax_send_sems, amax_recv_sems):
        my = lax.axis_index("i")
        pos = pos_ref[my]
        succ_a = cyc_ref[lax.rem(pos + 1, N_DEV)]
        pred_a = cyc_ref[lax.rem(pos - 1 + N_DEV, N_DEV)]
        succ_b = pred_a
        pred_b = succ_a

        barrier_sem = pltpu.get_barrier_semaphore()
        for nbr in (succ_a, pred_a):
            pl.semaphore_signal(barrier_sem, inc=1, device_id=(nbr,),
                                device_id_type=pl.DeviceIdType.MESH)
        pl.semaphore_wait(barrier_sem, 2)

        comm_a[0, :, :] = x_ref[:, :]
        comm_b[0, :, :] = x_ref[:, :]

        def gemm(src):
            return jnp.dot(src, w_ref[:, :],
                           preferred_element_type=jnp.float32,
                           precision=lax.Precision.HIGHEST)

        def store(origin, y):
            out_ref[pl.ds(origin * m_per, m_per), :] = y

        half = m_per // 2

        def make_hop(comm, send_sems, recv_sems, h, tgt, rows=None):
            src = comm.at[h % 2] if rows is None else comm.at[h % 2, rows]
            dst = (comm.at[(h + 1) % 2] if rows is None
                   else comm.at[(h + 1) % 2, rows])
            return pltpu.make_async_remote_copy(
                src_ref=src,
                dst_ref=dst,
                send_sem=send_sems.at[h % 2],
                recv_sem=recv_sems.at[(h + 1) % 2],
                device_id=(tgt,),
                device_id_type=pl.DeviceIdType.MESH,
            )

        amax = jnp.float32(0.0)
        for h in range(HOPS_A):
            last = h == HOPS_A - 1
            if h >= 1:
                pl.semaphore_wait(credit_a, 1)
            rdma_a = make_hop(comm_a, send_a, recv_a, h, succ_a,
                              rows=pl.ds(0, half) if last else None)
            rdma_a.start()
            if h >= 1:
                pl.semaphore_wait(credit_b, 1)
            rdma_b = make_hop(comm_b, send_b, recv_b, h, succ_b,
                              rows=pl.ds(half, half) if last else None)
            rdma_b.start()

            if h == 0:
                y = gemm(x_ref[:, :])
                store(my, y)
                amax = jnp.max(jnp.abs(y))
            else:
                y = gemm(comm_a[h % 2, :, :])
                store(cyc_ref[lax.rem(pos - h + N_DEV, N_DEV)], y)
                amax = jnp.maximum(amax, jnp.max(jnp.abs(y)))
                if h <= HOPS_B:
                    y = gemm(comm_b[h % 2, :, :])
                    store(cyc_ref[lax.rem(pos + h, N_DEV)], y)
                    amax = jnp.maximum(amax, jnp.max(jnp.abs(y)))

            rdma_a.wait_send()
            if h < HOPS_A - 1:
                pl.semaphore_signal(credit_a, inc=1, device_id=(pred_a,),
                                    device_id_type=pl.DeviceIdType.MESH)
            rdma_b.wait_send()
            if h < HOPS_B - 1:
                pl.semaphore_signal(credit_b, inc=1, device_id=(pred_b,),
                                    device_id_type=pl.DeviceIdType.MESH)
            rdma_a.wait_recv()
            rdma_b.wait_recv()

        anti = cyc_ref[lax.rem(pos + N_DEV // 2, N_DEV)]
        y = gemm(comm_a[HOPS_A % 2, :half, :])
        out_ref[pl.ds(anti * m_per, half), :] = y
        amax = jnp.maximum(amax, jnp.max(jnp.abs(y)))
        y = gemm(comm_b[HOPS_B % 2, half:, :])
        out_ref[pl.ds(anti * m_per + half, half), :] = y
        amax = jnp.maximum(amax, jnp.max(jnp.abs(y)))

        maxima_ref[pl.ds(my, 1), :] = jnp.full((1, 128), amax, jnp.float32)
        amax_rdmas = []
        for j in range(1, N_DEV):
            tgt = lax.rem(my + j, N_DEV)
            r = pltpu.make_async_remote_copy(
                src_ref=maxima_ref.at[pl.ds(my, 1)],
                dst_ref=maxima_ref.at[pl.ds(my, 1)],
                send_sem=amax_send_sems.at[j - 1],
                recv_sem=amax_recv_sems.at[j - 1],
                device_id=(tgt,),
                device_id_type=pl.DeviceIdType.MESH,
            )
            r.start()
            amax_rdmas.append(r)
        for r in amax_rdmas:
            r.wait_send()
        for r in amax_rdmas:
            r.wait_recv()

        gmax = jnp.max(maxima_ref[:, :])
        scale = gmax / 448.0
        out_ref[:, :] = _snap_e4m3(out_ref[:, :] / scale) * scale

    return pl.pallas_call(
        body,
        out_shape=jax.ShapeDtypeStruct((N_DEV * m_per, n_per), jnp.float32),
        in_specs=[
            pl.BlockSpec(memory_space=pltpu.SMEM),
            pl.BlockSpec(memory_space=pltpu.SMEM),
            pl.BlockSpec(memory_space=pltpu.VMEM),
            pl.BlockSpec(memory_space=pltpu.VMEM),
        ],
        out_specs=pl.BlockSpec(memory_space=pltpu.VMEM),
        scratch_shapes=[
            pltpu.VMEM((2, m_per, k), jnp.float32),
            pltpu.VMEM((2, m_per, k), jnp.float32),
            pltpu.VMEM((N_DEV, 128), jnp.float32),
            pltpu.SemaphoreType.DMA((2,)),
            pltpu.SemaphoreType.DMA((2,)),
            pltpu.SemaphoreType.DMA((2,)),
            pltpu.SemaphoreType.DMA((2,)),
            pltpu.SemaphoreType.REGULAR,
            pltpu.SemaphoreType.REGULAR,
            pltpu.SemaphoreType.DMA((N_DEV - 1,)),
            pltpu.SemaphoreType.DMA((N_DEV - 1,)),
        ],
        compiler_params=pltpu.CompilerParams(collective_id=0),
    )(jnp.asarray(A_CYCLE, jnp.int32), jnp.asarray(POS_A, jnp.int32),
      x, w_mat)


# device time: 304340 ns/iter; 2.5094x vs baseline; 1.2285x over previous
import jax
import jax.numpy as jnp
from jax import lax
from jax.experimental import pallas as pl
from jax.experimental.pallas import tpu as pltpu

N_DEV = 16
N_Z = 4
N_W = 4


def _snap_e4m3(v):
    q = jnp.clip(v, -448.0, 448.0).astype(jnp.float8_e4m3fn)
    return q.astype(jnp.float32)


def kernel(x, w_mat):
    m_per, k = x.shape
    _, n_per = w_mat.shape
    half = m_per // 2

    def body(x_ref, w_ref, out_ref,
             up_buf, down_buf, stage, from_prev, from_next, anti, maxima_ref,
             up_send, up_recv, down_send, down_recv,
             h1n_send, h1p_send, fp_recv, fn_recv,
             ft_send, fb_send, at_recv, ab_recv,
             cr_h1n, cr_h1p, cr_fwd_n, cr_fwd_p,
             amax_send_sems, amax_recv_sems):
        my = lax.axis_index("i")
        z = my // N_W
        w = my % N_W
        nxt = z * N_W + lax.rem(w + 1, N_W)
        prv = z * N_W + lax.rem(w - 1 + N_W, N_W)
        anti_w = z * N_W + lax.rem(w + 2, N_W)
        up = my + N_W
        down = my - N_W

        barrier_sem = pltpu.get_barrier_semaphore()
        for nbr in (nxt, prv):
            pl.semaphore_signal(barrier_sem, inc=1, device_id=(nbr,),
                                device_id_type=pl.DeviceIdType.MESH)

        @pl.when(z < N_Z - 1)
        def _():
            pl.semaphore_signal(barrier_sem, inc=1, device_id=(up,),
                                device_id_type=pl.DeviceIdType.MESH)

        @pl.when(z > 0)
        def _():
            pl.semaphore_signal(barrier_sem, inc=1, device_id=(down,),
                                device_id_type=pl.DeviceIdType.MESH)

        n_nbrs = 2 + (z > 0).astype(jnp.int32) + (z < N_Z - 1).astype(jnp.int32)
        pl.semaphore_wait(barrier_sem, n_nbrs)

        def up_step(s):
            return pltpu.make_async_remote_copy(
                src_ref=x_ref if s == 0 else up_buf.at[s - 1],
                dst_ref=up_buf.at[s],
                send_sem=up_send.at[s],
                recv_sem=up_recv.at[s],
                device_id=(up,),
                device_id_type=pl.DeviceIdType.MESH,
            )

        def down_step(s):
            return pltpu.make_async_remote_copy(
                src_ref=x_ref if s == 0 else down_buf.at[s - 1],
                dst_ref=down_buf.at[s],
                send_sem=down_send.at[s],
                recv_sem=down_recv.at[s],
                device_id=(down,),
                device_id_type=pl.DeviceIdType.MESH,
            )

        def up_send_cond(s):
            return (z >= s) & (z < N_Z - 1)

        def down_send_cond(s):
            return (z <= N_Z - 1 - s) & (z > 0)

        @pl.when(up_send_cond(0))
        def _():
            up_step(0).start()

        @pl.when(down_send_cond(0))
        def _():
            down_step(0).start()

        def gemm(src):
            return jnp.dot(src, w_ref[:, :],
                           preferred_element_type=jnp.float32,
                           precision=lax.Precision.HIGHEST)

        def store(origin, y):
            out_ref[pl.ds(origin * m_per, m_per), :] = y

        def phase1_interleave(qq):
            s = qq - 1

            @pl.when(z >= s + 1)
            def _():
                up_step(s).wait_recv()

            @pl.when(z <= 2 - s)
            def _():
                down_step(s).wait_recv()

            if s + 1 <= 2:
                @pl.when(up_send_cond(s + 1))
                def _():
                    up_step(s + 1).start()

                @pl.when(down_send_cond(s + 1))
                def _():
                    down_step(s + 1).start()

        def stage_copy(qq):
            if qq == 1:
                @pl.when(z == 0)
                def _():
                    stage[0, :, :] = down_buf[0, :, :]

                @pl.when(z >= 1)
                def _():
                    stage[0, :, :] = up_buf[0, :, :]

                return jnp.where(z == 0, 1, -1).astype(jnp.int32)
            if qq == 2:
                @pl.when(z == 0)
                def _():
                    stage[0, :, :] = down_buf[1, :, :]

                @pl.when((z == 1) | (z == 2))
                def _():
                    stage[0, :, :] = down_buf[0, :, :]

                @pl.when(z == 3)
                def _():
                    stage[0, :, :] = up_buf[1, :, :]

                return jnp.where(z == 0, 2,
                                 jnp.where(z == 3, -2, 1)).astype(jnp.int32)
            @pl.when(z == 0)
            def _():
                stage[0, :, :] = down_buf[2, :, :]

            @pl.when(z == 1)
            def _():
                stage[0, :, :] = down_buf[1, :, :]

            @pl.when(z == 2)
            def _():
                stage[0, :, :] = up_buf[1, :, :]

            @pl.when(z == 3)
            def _():
                stage[0, :, :] = up_buf[2, :, :]

            return jnp.where(z == 0, 3,
                             jnp.where(z == 1, 2,
                                       jnp.where(z == 2, -2, -3))
                             ).astype(jnp.int32)

        def h1_pair(qq):
            sl = qq % 2
            r_n = pltpu.make_async_remote_copy(
                src_ref=stage.at[0], dst_ref=from_prev.at[sl],
                send_sem=h1n_send.at[sl], recv_sem=fp_recv.at[sl],
                device_id=(nxt,), device_id_type=pl.DeviceIdType.MESH)
            r_p = pltpu.make_async_remote_copy(
                src_ref=stage.at[0], dst_ref=from_next.at[sl],
                send_sem=h1p_send.at[sl], recv_sem=fn_recv.at[sl],
                device_id=(prv,), device_id_type=pl.DeviceIdType.MESH)
            return r_n, r_p

        def fwd_pair(qq):
            sl = qq % 2
            r_t = pltpu.make_async_remote_copy(
                src_ref=from_prev.at[sl, pl.ds(0, half)],
                dst_ref=anti.at[sl, pl.ds(0, half)],
                send_sem=ft_send.at[sl], recv_sem=at_recv.at[sl],
                device_id=(nxt,), device_id_type=pl.DeviceIdType.MESH)
            r_b = pltpu.make_async_remote_copy(
                src_ref=from_next.at[sl, pl.ds(half, half)],
                dst_ref=anti.at[sl, pl.ds(half, half)],
                send_sem=fb_send.at[sl], recv_sem=ab_recv.at[sl],
                device_id=(prv,), device_id_type=pl.DeviceIdType.MESH)
            return r_t, r_b

        amax = jnp.float32(0.0)

        def maxup(a, y):
            return jnp.maximum(a, jnp.max(jnp.abs(y)))

        r_h1 = [None] * N_W
        r_fwd = [None] * N_W
        dzs = [None] * N_W

        stage[0, :, :] = x_ref[:, :]
        dzs[0] = jnp.int32(0)
        r_h1[0] = h1_pair(0)
        r_h1[0][0].start()
        r_h1[0][1].start()
        y = gemm(stage[0, :, :])
        store(my, y)
        amax = maxup(amax, y)

        for q in range(N_W):
            sl = q % 2
            if q >= 1:
                pt, pb = r_fwd[q - 1]
                pt.wait_recv()
                pb.wait_recv()
                y = gemm(anti[(q - 1) % 2, :, :])
                store(anti_w + N_W * dzs[q - 1], y)
                amax = maxup(amax, y)
                if q - 1 < 2:
                    pl.semaphore_signal(cr_fwd_n, inc=1, device_id=(prv,),
                                        device_id_type=pl.DeviceIdType.MESH)
                    pl.semaphore_signal(cr_fwd_p, inc=1, device_id=(nxt,),
                                        device_id_type=pl.DeviceIdType.MESH)

            rn, rp = r_h1[q]
            rn.wait_recv()
            rp.wait_recv()

            if q >= 2:
                pl.semaphore_wait(cr_fwd_n, 1)
                pl.semaphore_wait(cr_fwd_p, 1)
            r_fwd[q] = fwd_pair(q)
            r_fwd[q][0].start()
            r_fwd[q][1].start()

            y = gemm(from_prev[sl, :, :])
            store(prv + N_W * dzs[q], y)
            amax = maxup(amax, y)
            y = gemm(from_next[sl, :, :])
            store(nxt + N_W * dzs[q], y)
            amax = maxup(amax, y)

            r_fwd[q][0].wait_send()
            r_fwd[q][1].wait_send()
            if q < 2:
                pl.semaphore_signal(cr_h1n, inc=1, device_id=(prv,),
                                    device_id_type=pl.DeviceIdType.MESH)
                pl.semaphore_signal(cr_h1p, inc=1, device_id=(nxt,),
                                    device_id_type=pl.DeviceIdType.MESH)
            rn.wait_send()
            rp.wait_send()

            if q < N_W - 1:
                phase1_interleave(q + 1)
                dzs[q + 1] = stage_copy(q + 1)
                if q + 1 >= 2:
                    pl.semaphore_wait(cr_h1n, 1)
                    pl.semaphore_wait(cr_h1p, 1)
                r_h1[q + 1] = h1_pair(q + 1)
                r_h1[q + 1][0].start()
                r_h1[q + 1][1].start()
                y = gemm(stage[0, :, :])
                store(my + N_W * dzs[q + 1], y)
                amax = maxup(amax, y)

        pt, pb = r_fwd[N_W - 1]
        pt.wait_recv()
        pb.wait_recv()
        y = gemm(anti[(N_W - 1) % 2, :, :])
        store(anti_w + N_W * dzs[N_W - 1], y)
        amax = maxup(amax, y)

        for s in range(3):
            @pl.when(up_send_cond(s))
            def _(s=s):
                up_step(s).wait_send()

            @pl.when(down_send_cond(s))
            def _(s=s):
                down_step(s).wait_send()

        maxima_ref[pl.ds(my, 1), :] = jnp.full((1, 128), amax, jnp.float32)
        amax_rdmas = []
        for j in range(1, N_DEV):
            tgt = lax.rem(my + j, N_DEV)
            r = pltpu.make_async_remote_copy(
                src_ref=maxima_ref.at[pl.ds(my, 1)],
                dst_ref=maxima_ref.at[pl.ds(my, 1)],
                send_sem=amax_send_sems.at[j - 1],
                recv_sem=amax_recv_sems.at[j - 1],
                device_id=(tgt,),
                device_id_type=pl.DeviceIdType.MESH,
            )
            r.start()
            amax_rdmas.append(r)
        for r in amax_rdmas:
            r.wait_send()
        for r in amax_rdmas:
            r.wait_recv()

        gmax = jnp.max(maxima_ref[:, :])
        scale = gmax / 448.0
        out_ref[:, :] = _snap_e4m3(out_ref[:, :] / scale) * scale

    return pl.pallas_call(
        body,
        out_shape=jax.ShapeDtypeStruct((N_DEV * m_per, n_per), jnp.float32),
        in_specs=[
            pl.BlockSpec(memory_space=pltpu.VMEM),
            pl.BlockSpec(memory_space=pltpu.VMEM),
        ],
        out_specs=pl.BlockSpec(memory_space=pltpu.VMEM),
        scratch_shapes=[
            pltpu.VMEM((3, m_per, k), jnp.float32),
            pltpu.VMEM((3, m_per, k), jnp.float32),
            pltpu.VMEM((1, m_per, k), jnp.float32),
            pltpu.VMEM((2, m_per, k), jnp.float32),
            pltpu.VMEM((2, m_per, k), jnp.float32),
            pltpu.VMEM((2, m_per, k), jnp.float32),
            pltpu.VMEM((N_DEV, 128), jnp.float32),
            pltpu.SemaphoreType.DMA((3,)),
            pltpu.SemaphoreType.DMA((3,)),
            pltpu.SemaphoreType.DMA((3,)),
            pltpu.SemaphoreType.DMA((3,)),
            pltpu.SemaphoreType.DMA((2,)),
            pltpu.SemaphoreType.DMA((2,)),
            pltpu.SemaphoreType.DMA((2,)),
            pltpu.SemaphoreType.DMA((2,)),
            pltpu.SemaphoreType.DMA((2,)),
            pltpu.SemaphoreType.DMA((2,)),
            pltpu.SemaphoreType.DMA((2,)),
            pltpu.SemaphoreType.DMA((2,)),
            pltpu.SemaphoreType.REGULAR,
            pltpu.SemaphoreType.REGULAR,
            pltpu.SemaphoreType.REGULAR,
            pltpu.SemaphoreType.REGULAR,
            pltpu.SemaphoreType.DMA((N_DEV - 1,)),
            pltpu.SemaphoreType.DMA((N_DEV - 1,)),
        ],
        compiler_params=pltpu.CompilerParams(
            collective_id=0, vmem_limit_bytes=100 * 1024 * 1024),
    )(x, w_mat)
